# baseline (speedup 1.0000x reference)
import functools

import jax
import jax.numpy as jnp
from jax import lax
from jax.experimental import pallas as pl
from jax.experimental.pallas import tpu as pltpu
from jax.experimental.pallas import tpu_sc as plsc

D = 64
W = 128
SPLIT = (0, 96)
SIZES = (96, 104)
NBUF = 4
NC = 2
NS = 16
NW = NC * NS


@functools.lru_cache(maxsize=None)
def _build(batch: int, seq: int):
    rows_per_w = batch // NW
    assert rows_per_w % NBUF == 0 and seq == sum(SIZES)
    ngrp = rows_per_w // NBUF

    mesh = plsc.VectorSubcoreMesh(core_axis_name="c", subcore_axis_name="s")

    @functools.partial(
        pl.kernel,
        mesh=mesh,
        out_type=jax.ShapeDtypeStruct((batch, seq, W), jnp.float32),
        compiler_params=pltpu.CompilerParams(use_tc_tiling_on_sc=False),
        scratch_types=(
            [
                pltpu.VMEM((rows_per_w, seq), jnp.int32),
                pltpu.VMEM((NBUF, seq, W), jnp.float32),
            ]
            + [pltpu.SemaphoreType.DMA] * (2 * NBUF)
        ),
    )
    def run(x_hbm, table_hbm, out_hbm, idx_v, rows_v, *sems):
        sem_g = sems[:NBUF]
        sem_s = sems[NBUF:]
        wid = lax.axis_index("s") * NC + lax.axis_index("c")
        base = wid * rows_per_w
        pltpu.sync_copy(x_hbm.at[pl.ds(base, rows_per_w)], idx_v)

        def group(g, carry):
            for b in range(NBUF):

                @pl.when(g > 0)
                def _():
                    pltpu.make_async_copy(
                        rows_v.at[b], out_hbm.at[0], sem_s[b]
                    ).wait()

            gathers = []
            for b in range(NBUF):
                r = g * NBUF + b
                for off, sz in zip(SPLIT, SIZES):
                    gathers.append(
                        pltpu.async_copy(
                            table_hbm.at[idx_v.at[r, pl.ds(off, sz)]],
                            rows_v.at[b, pl.ds(off, sz)],
                            sem_g[b],
                        )
                    )
            for b in range(NBUF):
                r = g * NBUF + b
                gathers[2 * b].wait()
                gathers[2 * b + 1].wait()
                pltpu.async_copy(rows_v.at[b], out_hbm.at[base + r], sem_s[b])
            return carry

        lax.fori_loop(0, ngrp, group, 0)
        for b in range(NBUF):
            pltpu.make_async_copy(
                rows_v.at[b], out_hbm.at[0], sem_s[b]
            ).wait()

    return run


def kernel(x, table):
    B, S = x.shape
    V, _ = table.shape
    tp = jnp.broadcast_to(table[:, None, :], (V, W // D, D)).reshape(V, W)
    ko = _build(B, S)(x.astype(jnp.int32), tp)
    return ko[:, :, :D]


# speedup vs baseline: 1.0743x; 1.0743x over previous
import functools

import jax
import jax.numpy as jnp
from jax import lax
from jax.experimental import pallas as pl
from jax.experimental.pallas import tpu as pltpu
from jax.experimental.pallas import tpu_sc as plsc

D = 64
W = 128
SPLIT = (0, 96)
SIZES = (96, 104)
NBUF = 4
NC = 2
NS = 16
NW = NC * NS


@functools.lru_cache(maxsize=None)
def _build(batch: int, seq: int):
    rows_per_w = batch // NW
    assert rows_per_w % NBUF == 0 and seq == sum(SIZES)
    ngrp = rows_per_w // NBUF

    mesh = plsc.VectorSubcoreMesh(core_axis_name="c", subcore_axis_name="s")

    @functools.partial(
        pl.kernel,
        mesh=mesh,
        out_type=jax.ShapeDtypeStruct((batch, seq, W), jnp.float32),
        compiler_params=pltpu.CompilerParams(use_tc_tiling_on_sc=False),
        scratch_types=(
            [
                pltpu.VMEM((rows_per_w, seq), jnp.int32),
                pltpu.VMEM((NBUF, seq, W), jnp.float32),
            ]
            + [pltpu.SemaphoreType.DMA] * (2 * NBUF)
        ),
    )
    def run(x_hbm, table_hbm, out_hbm, idx_v, rows_v, *sems):
        sem_g = sems[:NBUF]
        sem_s = sems[NBUF:]
        wid = lax.axis_index("s") * NC + lax.axis_index("c")
        base = wid * rows_per_w
        pltpu.sync_copy(x_hbm.at[pl.ds(base, rows_per_w)], idx_v)

        def group(g, carry):
            for b in range(NBUF):

                @pl.when(g > 0)
                def _():
                    pltpu.make_async_copy(
                        rows_v.at[b], out_hbm.at[0], sem_s[b]
                    ).wait()

            gathers = []
            for b in range(NBUF):
                r = g * NBUF + b
                for off, sz in zip(SPLIT, SIZES):
                    gathers.append(
                        pltpu.async_copy(
                            table_hbm.at[idx_v.at[r, pl.ds(off, sz)]],
                            rows_v.at[b, pl.ds(off, sz)],
                            sem_g[b],
                        )
                    )
            for b in range(NBUF):
                r = g * NBUF + b
                gathers[2 * b].wait()
                gathers[2 * b + 1].wait()
                pltpu.async_copy(rows_v.at[b], out_hbm.at[base + r], sem_s[b])
            return carry

        lax.fori_loop(0, ngrp, group, 0)
        for b in range(NBUF):
            pltpu.make_async_copy(
                rows_v.at[b], out_hbm.at[0], sem_s[b]
            ).wait()

    return run


def kernel(x, table):
    B, S = x.shape
    V, _ = table.shape
    tp = jnp.pad(table, ((0, 0), (0, W - D)))
    ko = _build(B, S)(x.astype(jnp.int32), tp)
    return ko[:, :, :D]
